# baseline (speedup 1.0000x reference)
import dataclasses
import functools

import jax
import jax.numpy as jnp
from jax import lax
from jax.experimental import pallas as pl
from jax.experimental.pallas import tpu as pltpu
from jax.experimental.pallas import tpu_sc as plsc

_B, _NBOX, _NNEG = 4, 50, 100
_H, _W = 248, 216
_TY, _TX = _H // 8, 2
_TILE = 1024
_PLANE_W = _TY * _TX * _TILE
_NPOS = _B * _NBOX
_NBG = _B * _NNEG
_NPOS_PAD = 208
_NBG_PAD = 400
_NIDX = 3 * _NPOS_PAD + _NBG
_OFF_XS = 0
_OFF_YS = 208
_OFF_BX = 416
_OFF_BY = 816
_OFF_XG = 1216
_OFF_YG = 1424
_OFF_INV = 1632
_NPACK = 1648
_ALPHA = 0.25
_BETA_LOC = 2.0
_LN2 = 0.6931471805599453

_N_TABLE = 2 * _B * 2 * _PLANE_W


def _repack_kernel(loc_ref, clf_ref, rt_ref, ct_ref, gt_ref, anchor_ref,
                   tab_ref, pk_ref, planes_v, sem0, sem1):
    c0 = pltpu.make_async_copy(loc_ref.at[:, 0, 0:2], planes_v.at[0], sem0)
    c1 = pltpu.make_async_copy(clf_ref.at[:, 0, 0:2], planes_v.at[1], sem1)
    c0.start()
    c1.start()
    c0.wait()
    c1.wait()
    for a in range(2):
        for b in range(_B):
            for ch in range(2):
                tab_ref[a, b, ch, :, 0] = planes_v[a, b, ch, :, 0:128].reshape(
                    _TY, 8, 128)
                tab_ref[a, b, ch, :, 1, :, 0:_W - 128] = planes_v[
                    a, b, ch, :, 128:_W].reshape(_TY, 8, _W - 128)

    pk_ref[...] = jnp.zeros((_NPACK,), jnp.int32)
    pk_ref[_OFF_XS:_OFF_XS + _NPOS] = rt_ref[:, :, 0].reshape(_NPOS)
    pk_ref[_OFF_YS:_OFF_YS + _NPOS] = rt_ref[:, :, 1].reshape(_NPOS)
    pk_ref[_OFF_BX:_OFF_BX + _NBG] = ct_ref[:, :, 1].reshape(_NBG)
    pk_ref[_OFF_BY:_OFF_BY + _NBG] = ct_ref[:, :, 2].reshape(_NBG)
    g0 = gt_ref[:, :, 0].reshape(_NPOS)
    g1 = gt_ref[:, :, 1].reshape(_NPOS)
    g2 = gt_ref[:, :, 2].reshape(_NPOS)
    g3 = gt_ref[:, :, 3].reshape(_NPOS)
    x_gt = g0 + (g2 - g0) * 0.5
    y_gt = g1 - (g3 - g1) * 0.5
    pk_ref[_OFF_XG:_OFF_XG + _NPOS] = lax.bitcast_convert_type(
        x_gt, jnp.int32)
    pk_ref[_OFF_YG:_OFF_YG + _NPOS] = lax.bitcast_convert_type(
        y_gt, jnp.int32)
    a0 = anchor_ref[0]
    a1 = anchor_ref[1]
    inv_da = lax.rsqrt(a0 * a0 + a1 * a1)
    pk_ref[_OFF_INV:_OFF_INV + 16] = lax.bitcast_convert_type(
        jnp.broadcast_to(inv_da, (16,)), jnp.int32)


_repack = pl.pallas_call(
    _repack_kernel,
    out_shape=(
        jax.ShapeDtypeStruct((2, _B, 2, _TY, _TX, 8, 128), jnp.float32),
        jax.ShapeDtypeStruct((_NPACK,), jnp.int32),
    ),
    grid=(1,),
    in_specs=[
        pl.BlockSpec(memory_space=pl.ANY),
        pl.BlockSpec(memory_space=pl.ANY),
        pl.BlockSpec((_B, _NBOX, 2), lambda i: (0, 0, 0)),
        pl.BlockSpec((_B, _NNEG, 3), lambda i: (0, 0, 0)),
        pl.BlockSpec((_B, _NBOX, 4), lambda i: (0, 0, 0)),
        pl.BlockSpec((2,), lambda i: (0,)),
    ],
    out_specs=(
        pl.BlockSpec((2, _B, 2, _TY, _TX, 8, 128),
                     lambda i: (0, 0, 0, 0, 0, 0, 0)),
        pl.BlockSpec((_NPACK,), lambda i: (0,)),
    ),
    scratch_shapes=[
        pltpu.VMEM((2, _B, 2, _H, _W), jnp.float32),
        pltpu.SemaphoreType.DMA,
        pltpu.SemaphoreType.DMA,
    ],
)


def _ln(p):
    bits = lax.bitcast_convert_type(p, jnp.int32)
    e = jnp.right_shift(bits, 23) - 127
    m = lax.bitcast_convert_type(
        jnp.bitwise_or(jnp.bitwise_and(bits, 0x007FFFFF), 0x3F800000),
        jnp.float32)
    t = (m - 1.0) / (m + 1.0)
    t2 = t * t
    ln_m = t * (2.0 + t2 * (2.0 / 3.0 + t2 * (2.0 / 5.0
                + t2 * (2.0 / 7.0 + t2 * (2.0 / 9.0)))))
    return e.astype(jnp.float32) * _LN2 + ln_m


def _focal(p):
    one_m = 1.0 - p
    return -_ln(p) * (_ALPHA * one_m * one_m)


def _huber(x):
    ax = jnp.abs(x)
    return jnp.where(ax < 1.0, 0.5 * x * x, ax - 0.5)


def _tile_word(b, y, x):
    plane = b * 2
    tile = (plane * _TY + jnp.right_shift(y, 3)) * _TX + jnp.right_shift(x, 7)
    return (tile * _TILE + jnp.left_shift(jnp.bitwise_and(y, 7), 7)
            + jnp.bitwise_and(x, 127))


_mesh = plsc.VectorSubcoreMesh(core_axis_name="c", subcore_axis_name="s")

_cp = pltpu.CompilerParams()
if "needs_layout_passes" in pltpu.CompilerParams.__dataclass_fields__:
    _cp = dataclasses.replace(_cp, needs_layout_passes=False)


@functools.partial(
    pl.kernel,
    out_type=jax.ShapeDtypeStruct((16,), jnp.float32),
    mesh=_mesh,
    compiler_params=_cp,
    scratch_types=[
        pltpu.VMEM((_NPACK,), jnp.int32),
        pltpu.VMEM((_NIDX,), jnp.int32),
        pltpu.VMEM((_NIDX,), jnp.float32),
        pltpu.VMEM((16,), jnp.float32),
        pltpu.SemaphoreType.DMA,
    ],
)
def _loss_kernel(tab_hbm, pk_hbm, out_hbm,
                 pk_v, idx_v, val_v, out_v, sem):
    cid = lax.axis_index("c")
    sid = lax.axis_index("s")

    @pl.when(jnp.logical_and(cid == 0, sid == 0))
    def _():
        pltpu.sync_copy(pk_hbm, pk_v)

        lanes = lax.iota(jnp.int32, 16)

        for i in range(_NPOS_PAD // 16):
            p = lanes + (i * 16)
            x = pk_v[pl.ds(_OFF_XS + i * 16, 16)]
            y = pk_v[pl.ds(_OFF_YS + i * 16, 16)]
            b = (jnp.where(p >= _NBOX, 1, 0)
                 + jnp.where(p >= 2 * _NBOX, 1, 0)
                 + jnp.where(p >= 3 * _NBOX, 1, 0))
            base = _tile_word(b, y, x)
            idx_v[pl.ds(i * 16, 16)] = base
            idx_v[pl.ds(_NPOS_PAD + i * 16, 16)] = base + _PLANE_W
            idx_v[pl.ds(2 * _NPOS_PAD + i * 16, 16)] = base + 9 * _PLANE_W

        copies = [pltpu.async_copy(tab_hbm.at[idx_v.at[pl.ds(off, 128)]],
                                   val_v.at[pl.ds(off, 128)], sem)
                  for off in range(0, 512, 128)]

        for i in range(_NBG_PAD // 16):
            q = lanes + (i * 16)
            bx = pk_v[pl.ds(_OFF_BX + i * 16, 16)]
            by = pk_v[pl.ds(_OFF_BY + i * 16, 16)]
            b = (jnp.where(q >= _NNEG, 1, 0)
                 + jnp.where(q >= 2 * _NNEG, 1, 0)
                 + jnp.where(q >= 3 * _NNEG, 1, 0))
            idx_v[pl.ds(3 * _NPOS_PAD + i * 16, 16)] = (
                _tile_word(b, by, bx) + 8 * _PLANE_W)

        copies += [pltpu.async_copy(tab_hbm.at[idx_v.at[pl.ds(off, 128)]],
                                    val_v.at[pl.ds(off, 128)], sem)
                   for off in range(512, _NIDX, 128)]

        inv_da = plsc.bitcast(pk_v[pl.ds(_OFF_INV, 16)], jnp.float32)

        for c in copies:
            c.wait()

        sl_acc = jnp.zeros((16,), jnp.float32)
        car_acc = jnp.zeros((16,), jnp.float32)
        for i in range(_NPOS_PAD // 16):
            p = lanes + (i * 16)
            w = jnp.where(p < _NPOS, 1.0, 0.0)
            x_gt = plsc.bitcast(pk_v[pl.ds(_OFF_XG + i * 16, 16)],
                                jnp.float32)
            y_gt = plsc.bitcast(pk_v[pl.ds(_OFF_YG + i * 16, 16)],
                                jnp.float32)
            dx = (x_gt - val_v[pl.ds(i * 16, 16)]) * inv_da
            dy = (y_gt - val_v[pl.ds(_NPOS_PAD + i * 16, 16)]) * inv_da
            sl_acc = sl_acc + w * (_huber(dx) + _huber(dy))
            car_acc = car_acc + w * _focal(
                val_v[pl.ds(2 * _NPOS_PAD + i * 16, 16)])

        bg_acc = jnp.zeros((16,), jnp.float32)
        for i in range(_NBG_PAD // 16):
            bg_acc = bg_acc + _focal(val_v[pl.ds(3 * _NPOS_PAD + i * 16, 16)])

        tot = (sl_acc * (_BETA_LOC / _NPOS)
               + car_acc * (1.0 / ((_B - 1) * (_NBOX - 1)))
               + bg_acc * (1.0 / ((_B - 1) * (_NNEG - 1))))
        out_v[...] = jnp.zeros((16,), jnp.float32) + jnp.sum(tot)
        pltpu.sync_copy(out_v, out_hbm)


def kernel(regression_targets, classification_targets_dict, gt_boxes_tensor,
           loc, size, clf, occupancy, angle, heading, anchor):
    rt = regression_targets.astype(jnp.int32)
    ct = classification_targets_dict.astype(jnp.int32)
    table, packed = _repack(loc, clf, rt, ct,
                            gt_boxes_tensor.astype(jnp.float32),
                            anchor.astype(jnp.float32))
    out = _loss_kernel(table.reshape(-1), packed)
    return out[0]


# speedup vs baseline: 1.3631x; 1.3631x over previous
import dataclasses
import functools

import jax
import jax.numpy as jnp
from jax import lax
from jax.experimental import pallas as pl
from jax.experimental.pallas import tpu as pltpu
from jax.experimental.pallas import tpu_sc as plsc

_B, _NBOX, _NNEG = 4, 50, 100
_H, _W = 248, 216
_TY, _TX = _W // 8, 2
_TILE = 1024
_PLANE_W = _TY * _TX * _TILE
_NPOS = _B * _NBOX
_NBG = _B * _NNEG
_NPOS_PAD = 208
_NBG_PAD = 400
_NIDX = 3 * _NPOS_PAD + _NBG
_OFF_XS = 0
_OFF_YS = 208
_OFF_BX = 416
_OFF_BY = 816
_OFF_XG = 1216
_OFF_YG = 1424
_OFF_INV = 1632
_NPACK = 1648
_ALPHA = 0.25
_BETA_LOC = 2.0
_LN2 = 0.6931471805599453


def _repack_kernel(loc_ref, clf_ref, rt_ref, ct_ref, gt_ref, anchor_ref,
                   tab_ref, pk_ref, planes_v, sem0, sem1):
    c0 = pltpu.make_async_copy(loc_ref.at[:, 0, 0:2], planes_v.at[0], sem0)
    c1 = pltpu.make_async_copy(clf_ref.at[:, 0, 0:2], planes_v.at[1], sem1)
    c0.start()
    c1.start()
    c0.wait()
    c1.wait()
    for a in range(2):
        for b in range(_B):
            for ch in range(2):
                tab_ref[a, b, ch, :, 0] = planes_v[a, b, ch, :, 0:128].reshape(
                    _TY, 8, 128)
                tab_ref[a, b, ch, :, 1, :, 0:_H - 128] = planes_v[
                    a, b, ch, :, 128:_H].reshape(_TY, 8, _H - 128)

    pk_ref[...] = jnp.zeros((_NPACK,), jnp.int32)
    pk_ref[_OFF_XS:_OFF_XS + _NPOS] = rt_ref[:, :, 0].reshape(_NPOS)
    pk_ref[_OFF_YS:_OFF_YS + _NPOS] = rt_ref[:, :, 1].reshape(_NPOS)
    pk_ref[_OFF_BX:_OFF_BX + _NBG] = ct_ref[:, :, 1].reshape(_NBG)
    pk_ref[_OFF_BY:_OFF_BY + _NBG] = ct_ref[:, :, 2].reshape(_NBG)
    g0 = gt_ref[:, :, 0].reshape(_NPOS)
    g1 = gt_ref[:, :, 1].reshape(_NPOS)
    g2 = gt_ref[:, :, 2].reshape(_NPOS)
    g3 = gt_ref[:, :, 3].reshape(_NPOS)
    x_gt = g0 + (g2 - g0) * 0.5
    y_gt = g1 - (g3 - g1) * 0.5
    pk_ref[_OFF_XG:_OFF_XG + _NPOS] = lax.bitcast_convert_type(
        x_gt, jnp.int32)
    pk_ref[_OFF_YG:_OFF_YG + _NPOS] = lax.bitcast_convert_type(
        y_gt, jnp.int32)
    a0 = anchor_ref[0]
    a1 = anchor_ref[1]
    inv_da = lax.rsqrt(a0 * a0 + a1 * a1)
    pk_ref[_OFF_INV:_OFF_INV + 16] = lax.bitcast_convert_type(
        jnp.broadcast_to(inv_da, (16,)), jnp.int32)


_repack = pl.pallas_call(
    _repack_kernel,
    out_shape=(
        jax.ShapeDtypeStruct((2, _B, 2, _TY, _TX, 8, 128), jnp.float32),
        jax.ShapeDtypeStruct((_NPACK,), jnp.int32),
    ),
    grid=(1,),
    in_specs=[
        pl.BlockSpec(memory_space=pl.ANY),
        pl.BlockSpec(memory_space=pl.ANY),
        pl.BlockSpec((_B, _NBOX, 2), lambda i: (0, 0, 0)),
        pl.BlockSpec((_B, _NNEG, 3), lambda i: (0, 0, 0)),
        pl.BlockSpec((_B, _NBOX, 4), lambda i: (0, 0, 0)),
        pl.BlockSpec((2,), lambda i: (0,)),
    ],
    out_specs=(
        pl.BlockSpec((2, _B, 2, _TY, _TX, 8, 128),
                     lambda i: (0, 0, 0, 0, 0, 0, 0)),
        pl.BlockSpec((_NPACK,), lambda i: (0,)),
    ),
    scratch_shapes=[
        pltpu.VMEM((2, _B, 2, _W, _H), jnp.float32),
        pltpu.SemaphoreType.DMA,
        pltpu.SemaphoreType.DMA,
    ],
)


def _ln(p):
    bits = lax.bitcast_convert_type(p, jnp.int32)
    e = jnp.right_shift(bits, 23) - 127
    m = lax.bitcast_convert_type(
        jnp.bitwise_or(jnp.bitwise_and(bits, 0x007FFFFF), 0x3F800000),
        jnp.float32)
    t = (m - 1.0) / (m + 1.0)
    t2 = t * t
    ln_m = t * (2.0 + t2 * (2.0 / 3.0 + t2 * (2.0 / 5.0
                + t2 * (2.0 / 7.0 + t2 * (2.0 / 9.0)))))
    return e.astype(jnp.float32) * _LN2 + ln_m


def _focal(p):
    one_m = 1.0 - p
    return -_ln(p) * (_ALPHA * one_m * one_m)


def _huber(x):
    ax = jnp.abs(x)
    return jnp.where(ax < 1.0, 0.5 * x * x, ax - 0.5)


def _tile_word(b, y, x):
    plane = b * 2
    tile = (plane * _TY + jnp.right_shift(x, 3)) * _TX + jnp.right_shift(y, 7)
    return (tile * _TILE + jnp.left_shift(jnp.bitwise_and(x, 7), 7)
            + jnp.bitwise_and(y, 127))


_mesh = plsc.VectorSubcoreMesh(core_axis_name="c", subcore_axis_name="s")

_cp = pltpu.CompilerParams()
if "needs_layout_passes" in pltpu.CompilerParams.__dataclass_fields__:
    _cp = dataclasses.replace(_cp, needs_layout_passes=False)


@functools.partial(
    pl.kernel,
    out_type=jax.ShapeDtypeStruct((16,), jnp.float32),
    mesh=_mesh,
    compiler_params=_cp,
    scratch_types=[
        pltpu.VMEM((_NPACK,), jnp.int32),
        pltpu.VMEM((_NIDX,), jnp.int32),
        pltpu.VMEM((_NIDX,), jnp.float32),
        pltpu.VMEM((16,), jnp.float32),
        pltpu.SemaphoreType.DMA,
    ],
)
def _loss_kernel(tab_hbm, pk_hbm, out_hbm,
                 pk_v, idx_v, val_v, out_v, sem):
    cid = lax.axis_index("c")
    sid = lax.axis_index("s")

    @pl.when(jnp.logical_and(cid == 0, sid == 0))
    def _():
        pltpu.sync_copy(pk_hbm, pk_v)

        lanes = lax.iota(jnp.int32, 16)

        for i in range(_NPOS_PAD // 16):
            p = lanes + (i * 16)
            x = pk_v[pl.ds(_OFF_XS + i * 16, 16)]
            y = pk_v[pl.ds(_OFF_YS + i * 16, 16)]
            b = (jnp.where(p >= _NBOX, 1, 0)
                 + jnp.where(p >= 2 * _NBOX, 1, 0)
                 + jnp.where(p >= 3 * _NBOX, 1, 0))
            base = _tile_word(b, y, x)
            idx_v[pl.ds(i * 16, 16)] = base
            idx_v[pl.ds(_NPOS_PAD + i * 16, 16)] = base + _PLANE_W
            idx_v[pl.ds(2 * _NPOS_PAD + i * 16, 16)] = base + 9 * _PLANE_W

        copies = [pltpu.async_copy(tab_hbm.at[idx_v.at[pl.ds(off, 128)]],
                                   val_v.at[pl.ds(off, 128)], sem)
                  for off in range(0, 512, 128)]

        for i in range(_NBG_PAD // 16):
            q = lanes + (i * 16)
            bx = pk_v[pl.ds(_OFF_BX + i * 16, 16)]
            by = pk_v[pl.ds(_OFF_BY + i * 16, 16)]
            b = (jnp.where(q >= _NNEG, 1, 0)
                 + jnp.where(q >= 2 * _NNEG, 1, 0)
                 + jnp.where(q >= 3 * _NNEG, 1, 0))
            idx_v[pl.ds(3 * _NPOS_PAD + i * 16, 16)] = (
                _tile_word(b, by, bx) + 8 * _PLANE_W)

        copies += [pltpu.async_copy(tab_hbm.at[idx_v.at[pl.ds(off, 128)]],
                                    val_v.at[pl.ds(off, 128)], sem)
                   for off in range(512, _NIDX, 128)]

        inv_da = plsc.bitcast(pk_v[pl.ds(_OFF_INV, 16)], jnp.float32)

        for c in copies:
            c.wait()

        sl_acc = jnp.zeros((16,), jnp.float32)
        car_acc = jnp.zeros((16,), jnp.float32)
        for i in range(_NPOS_PAD // 16):
            p = lanes + (i * 16)
            w = jnp.where(p < _NPOS, 1.0, 0.0)
            x_gt = plsc.bitcast(pk_v[pl.ds(_OFF_XG + i * 16, 16)],
                                jnp.float32)
            y_gt = plsc.bitcast(pk_v[pl.ds(_OFF_YG + i * 16, 16)],
                                jnp.float32)
            dx = (x_gt - val_v[pl.ds(i * 16, 16)]) * inv_da
            dy = (y_gt - val_v[pl.ds(_NPOS_PAD + i * 16, 16)]) * inv_da
            sl_acc = sl_acc + w * (_huber(dx) + _huber(dy))
            car_acc = car_acc + w * _focal(
                val_v[pl.ds(2 * _NPOS_PAD + i * 16, 16)])

        bg_acc = jnp.zeros((16,), jnp.float32)
        for i in range(_NBG_PAD // 16):
            bg_acc = bg_acc + _focal(val_v[pl.ds(3 * _NPOS_PAD + i * 16, 16)])

        tot = (sl_acc * (_BETA_LOC / _NPOS)
               + car_acc * (1.0 / ((_B - 1) * (_NBOX - 1)))
               + bg_acc * (1.0 / ((_B - 1) * (_NNEG - 1))))
        out_v[...] = jnp.zeros((16,), jnp.float32) + jnp.sum(tot)
        pltpu.sync_copy(out_v, out_hbm)


def kernel(regression_targets, classification_targets_dict, gt_boxes_tensor,
           loc, size, clf, occupancy, angle, heading, anchor):
    loc_t = jnp.swapaxes(loc, 3, 4)
    clf_t = jnp.swapaxes(clf, 3, 4)
    table, packed = _repack(loc_t, clf_t,
                            regression_targets.astype(jnp.int32),
                            classification_targets_dict.astype(jnp.int32),
                            gt_boxes_tensor.astype(jnp.float32),
                            anchor.astype(jnp.float32))
    out = _loss_kernel(table.reshape(-1), packed)
    return out[0]


# speedup vs baseline: 1.6741x; 1.2282x over previous
import dataclasses
import functools

import jax
import jax.numpy as jnp
from jax import lax
from jax.experimental import pallas as pl
from jax.experimental.pallas import tpu as pltpu
from jax.experimental.pallas import tpu_sc as plsc

_B, _NBOX, _NNEG = 4, 50, 100
_H, _W = 248, 216
_TY, _TX = _W // 8, 2
_TILE = 1024
_PLANE_W = _TY * _TX * _TILE
_NPOS = _B * _NBOX
_NBG = _B * _NNEG
_NPOS_PAD = 208
_NBG_PAD = 400
_NIDX = 3 * _NPOS_PAD + _NBG
_OFF_XS = 0
_OFF_YS = 208
_OFF_BX = 416
_OFF_BY = 816
_OFF_XG = 1216
_OFF_YG = 1424
_OFF_INV = 1632
_NPACK = 1648
_ALPHA = 0.25
_BETA_LOC = 2.0
_LN2 = 0.6931471805599453


def _repack_kernel(loc_ref, clf_ref, rt_ref, ct_ref, gt_ref, anchor_ref,
                   tab_ref, pk_ref, planes_v, sem0, sem1):
    copies = []
    for a, ref in enumerate((loc_ref, clf_ref)):
        for b in range(_B):
            copies.append(pltpu.make_async_copy(
                ref.at[b, 0, 0:2], planes_v.at[a, b],
                sem0 if a == 0 else sem1))
    for c in copies:
        c.start()
    zpad = jnp.zeros((8,), jnp.int32)
    pk_ref[_OFF_XS + _NPOS:_OFF_XS + _NPOS_PAD] = zpad
    pk_ref[_OFF_YS + _NPOS:_OFF_YS + _NPOS_PAD] = zpad
    pk_ref[_OFF_XG + _NPOS:_OFF_XG + _NPOS_PAD] = zpad
    pk_ref[_OFF_YG + _NPOS:_OFF_YG + _NPOS_PAD] = zpad
    for b in range(_B):
        pk_ref[pl.ds(_OFF_XS + b * _NBOX, _NBOX)] = rt_ref[b, 0]
        pk_ref[pl.ds(_OFF_YS + b * _NBOX, _NBOX)] = rt_ref[b, 1]
        pk_ref[pl.ds(_OFF_BX + b * _NNEG, _NNEG)] = ct_ref[1, b]
        pk_ref[pl.ds(_OFF_BY + b * _NNEG, _NNEG)] = ct_ref[2, b]
        g0 = gt_ref[b, 0]
        g1 = gt_ref[b, 1]
        g2 = gt_ref[b, 2]
        g3 = gt_ref[b, 3]
        pk_ref[pl.ds(_OFF_XG + b * _NBOX, _NBOX)] = lax.bitcast_convert_type(
            g0 + (g2 - g0) * 0.5, jnp.int32)
        pk_ref[pl.ds(_OFF_YG + b * _NBOX, _NBOX)] = lax.bitcast_convert_type(
            g1 - (g3 - g1) * 0.5, jnp.int32)
    a0 = anchor_ref[0]
    a1 = anchor_ref[1]
    inv_da = lax.rsqrt(a0 * a0 + a1 * a1)
    pk_ref[_OFF_INV:_OFF_INV + 16] = lax.bitcast_convert_type(
        jnp.broadcast_to(inv_da, (16,)), jnp.int32)
    for c in copies:
        c.wait()
    for a in range(2):
        for b in range(_B):
            for ch in range(2):
                tab_ref[a, b, ch, :, 0] = planes_v[a, b, ch, :, 0:128].reshape(
                    _TY, 8, 128)
                tab_ref[a, b, ch, :, 1, :, 0:_H - 128] = planes_v[
                    a, b, ch, :, 128:_H].reshape(_TY, 8, _H - 128)


_repack = pl.pallas_call(
    _repack_kernel,
    out_shape=(
        jax.ShapeDtypeStruct((2, _B, 2, _TY, _TX, 8, 128), jnp.float32),
        jax.ShapeDtypeStruct((_NPACK,), jnp.int32),
    ),
    grid=(1,),
    in_specs=[
        pl.BlockSpec(memory_space=pl.ANY),
        pl.BlockSpec(memory_space=pl.ANY),
        pl.BlockSpec((_B, 2, _NBOX), lambda i: (0, 0, 0)),
        pl.BlockSpec((3, _B, _NNEG), lambda i: (0, 0, 0)),
        pl.BlockSpec((_B, 4, _NBOX), lambda i: (0, 0, 0)),
        pl.BlockSpec((2,), lambda i: (0,)),
    ],
    out_specs=(
        pl.BlockSpec((2, _B, 2, _TY, _TX, 8, 128),
                     lambda i: (0, 0, 0, 0, 0, 0, 0)),
        pl.BlockSpec((_NPACK,), lambda i: (0,)),
    ),
    scratch_shapes=[
        pltpu.VMEM((2, _B, 2, _W, _H), jnp.float32),
        pltpu.SemaphoreType.DMA,
        pltpu.SemaphoreType.DMA,
    ],
)


def _ln(p):
    bits = lax.bitcast_convert_type(p, jnp.int32)
    e = jnp.right_shift(bits, 23) - 127
    m = lax.bitcast_convert_type(
        jnp.bitwise_or(jnp.bitwise_and(bits, 0x007FFFFF), 0x3F800000),
        jnp.float32)
    t = (m - 1.0) / (m + 1.0)
    t2 = t * t
    ln_m = t * (2.0 + t2 * (2.0 / 3.0 + t2 * (2.0 / 5.0
                + t2 * (2.0 / 7.0 + t2 * (2.0 / 9.0)))))
    return e.astype(jnp.float32) * _LN2 + ln_m


def _focal(p):
    one_m = 1.0 - p
    return -_ln(p) * (_ALPHA * one_m * one_m)


def _huber(x):
    ax = jnp.abs(x)
    return jnp.where(ax < 1.0, 0.5 * x * x, ax - 0.5)


def _tile_word(b, y, x):
    plane = b * 2
    tile = (plane * _TY + jnp.right_shift(x, 3)) * _TX + jnp.right_shift(y, 7)
    return (tile * _TILE + jnp.left_shift(jnp.bitwise_and(x, 7), 7)
            + jnp.bitwise_and(y, 127))


_mesh = plsc.VectorSubcoreMesh(core_axis_name="c", subcore_axis_name="s")

_cp = pltpu.CompilerParams()
if "needs_layout_passes" in pltpu.CompilerParams.__dataclass_fields__:
    _cp = dataclasses.replace(_cp, needs_layout_passes=False)


@functools.partial(
    pl.kernel,
    out_type=jax.ShapeDtypeStruct((16,), jnp.float32),
    mesh=_mesh,
    compiler_params=_cp,
    scratch_types=[
        pltpu.VMEM((_NPACK,), jnp.int32),
        pltpu.VMEM((_NIDX,), jnp.int32),
        pltpu.VMEM((_NIDX,), jnp.float32),
        pltpu.VMEM((16,), jnp.float32),
        pltpu.SemaphoreType.DMA,
    ],
)
def _loss_kernel(tab_hbm, pk_hbm, out_hbm,
                 pk_v, idx_v, val_v, out_v, sem):
    cid = lax.axis_index("c")
    sid = lax.axis_index("s")

    @pl.when(jnp.logical_and(cid == 0, sid == 0))
    def _():
        pltpu.sync_copy(pk_hbm, pk_v)

        lanes = lax.iota(jnp.int32, 16)

        for i in range(_NPOS_PAD // 16):
            p = lanes + (i * 16)
            x = pk_v[pl.ds(_OFF_XS + i * 16, 16)]
            y = pk_v[pl.ds(_OFF_YS + i * 16, 16)]
            b = (jnp.where(p >= _NBOX, 1, 0)
                 + jnp.where(p >= 2 * _NBOX, 1, 0)
                 + jnp.where(p >= 3 * _NBOX, 1, 0))
            base = _tile_word(b, y, x)
            idx_v[pl.ds(i * 16, 16)] = base
            idx_v[pl.ds(_NPOS_PAD + i * 16, 16)] = base + _PLANE_W
            idx_v[pl.ds(2 * _NPOS_PAD + i * 16, 16)] = base + 9 * _PLANE_W

        copies = [pltpu.async_copy(tab_hbm.at[idx_v.at[pl.ds(off, 128)]],
                                   val_v.at[pl.ds(off, 128)], sem)
                  for off in range(0, 512, 128)]

        for i in range(_NBG_PAD // 16):
            q = lanes + (i * 16)
            bx = pk_v[pl.ds(_OFF_BX + i * 16, 16)]
            by = pk_v[pl.ds(_OFF_BY + i * 16, 16)]
            b = (jnp.where(q >= _NNEG, 1, 0)
                 + jnp.where(q >= 2 * _NNEG, 1, 0)
                 + jnp.where(q >= 3 * _NNEG, 1, 0))
            idx_v[pl.ds(3 * _NPOS_PAD + i * 16, 16)] = (
                _tile_word(b, by, bx) + 8 * _PLANE_W)

        copies += [pltpu.async_copy(tab_hbm.at[idx_v.at[pl.ds(off, 128)]],
                                    val_v.at[pl.ds(off, 128)], sem)
                   for off in range(512, _NIDX, 128)]

        inv_da = plsc.bitcast(pk_v[pl.ds(_OFF_INV, 16)], jnp.float32)

        for c in copies:
            c.wait()

        sl_acc = jnp.zeros((16,), jnp.float32)
        car_acc = jnp.zeros((16,), jnp.float32)
        for i in range(_NPOS_PAD // 16):
            p = lanes + (i * 16)
            w = jnp.where(p < _NPOS, 1.0, 0.0)
            x_gt = plsc.bitcast(pk_v[pl.ds(_OFF_XG + i * 16, 16)],
                                jnp.float32)
            y_gt = plsc.bitcast(pk_v[pl.ds(_OFF_YG + i * 16, 16)],
                                jnp.float32)
            dx = (x_gt - val_v[pl.ds(i * 16, 16)]) * inv_da
            dy = (y_gt - val_v[pl.ds(_NPOS_PAD + i * 16, 16)]) * inv_da
            sl_acc = sl_acc + w * (_huber(dx) + _huber(dy))
            car_acc = car_acc + w * _focal(
                val_v[pl.ds(2 * _NPOS_PAD + i * 16, 16)])

        bg_acc = jnp.zeros((16,), jnp.float32)
        for i in range(_NBG_PAD // 16):
            bg_acc = bg_acc + _focal(val_v[pl.ds(3 * _NPOS_PAD + i * 16, 16)])

        tot = (sl_acc * (_BETA_LOC / _NPOS)
               + car_acc * (1.0 / ((_B - 1) * (_NBOX - 1)))
               + bg_acc * (1.0 / ((_B - 1) * (_NNEG - 1))))
        out_v[...] = jnp.zeros((16,), jnp.float32) + jnp.sum(tot)
        pltpu.sync_copy(out_v, out_hbm)


def kernel(regression_targets, classification_targets_dict, gt_boxes_tensor,
           loc, size, clf, occupancy, angle, heading, anchor):
    loc_t = jnp.swapaxes(loc, 3, 4)
    clf_t = jnp.swapaxes(clf, 3, 4)
    rt_t = jnp.swapaxes(regression_targets.astype(jnp.int32), 1, 2)
    ct_t = jnp.transpose(classification_targets_dict.astype(jnp.int32),
                         (2, 0, 1))
    gt_t = jnp.swapaxes(gt_boxes_tensor.astype(jnp.float32), 1, 2)
    table, packed = _repack(loc_t, clf_t, rt_t, ct_t, gt_t,
                            anchor.astype(jnp.float32))
    out = _loss_kernel(table.reshape(-1), packed)
    return out[0]


# speedup vs baseline: 1.6969x; 1.0136x over previous
import dataclasses
import functools

import jax
import jax.numpy as jnp
from jax import lax
from jax.experimental import pallas as pl
from jax.experimental.pallas import tpu as pltpu
from jax.experimental.pallas import tpu_sc as plsc

_B, _NBOX, _NNEG = 4, 50, 100
_H, _W = 248, 216
_TY, _TX = _W // 8, 2
_TILE = 1024
_PLANE_W = _TY * _TX * _TILE
_NPOS = _B * _NBOX
_NBG = _B * _NNEG
_NPOS_PAD = 208
_NBG_PAD = 400
_NIDX = 3 * _NPOS_PAD + _NBG
_OFF_XS = 0
_OFF_YS = 208
_OFF_BX = 416
_OFF_BY = 816
_OFF_XG = 1216
_OFF_YG = 1424
_OFF_INV = 1632
_NPACK = 1648
_ALPHA = 0.25
_BETA_LOC = 2.0
_LN2 = 0.6931471805599453


def _repack_kernel(loc_ref, clf_ref, rt_ref, ct_ref, gt_ref, anchor_ref,
                   tab_ref, pk_ref, planes_v, stag_v, sems, sem_out):
    copies = []
    for a, ref in enumerate((loc_ref, clf_ref)):
        for b in range(_B):
            copies.append(pltpu.make_async_copy(
                ref.at[b, 0, 0:2], planes_v.at[a, b],
                sems.at[a * _B + b]))
    for c in copies:
        c.start()
    zpad = jnp.zeros((8,), jnp.int32)
    pk_ref[_OFF_XS + _NPOS:_OFF_XS + _NPOS_PAD] = zpad
    pk_ref[_OFF_YS + _NPOS:_OFF_YS + _NPOS_PAD] = zpad
    pk_ref[_OFF_XG + _NPOS:_OFF_XG + _NPOS_PAD] = zpad
    pk_ref[_OFF_YG + _NPOS:_OFF_YG + _NPOS_PAD] = zpad
    for b in range(_B):
        pk_ref[pl.ds(_OFF_XS + b * _NBOX, _NBOX)] = rt_ref[b, 0]
        pk_ref[pl.ds(_OFF_YS + b * _NBOX, _NBOX)] = rt_ref[b, 1]
        pk_ref[pl.ds(_OFF_BX + b * _NNEG, _NNEG)] = ct_ref[1, b]
        pk_ref[pl.ds(_OFF_BY + b * _NNEG, _NNEG)] = ct_ref[2, b]
        g0 = gt_ref[b, 0]
        g1 = gt_ref[b, 1]
        g2 = gt_ref[b, 2]
        g3 = gt_ref[b, 3]
        pk_ref[pl.ds(_OFF_XG + b * _NBOX, _NBOX)] = lax.bitcast_convert_type(
            g0 + (g2 - g0) * 0.5, jnp.int32)
        pk_ref[pl.ds(_OFF_YG + b * _NBOX, _NBOX)] = lax.bitcast_convert_type(
            g1 - (g3 - g1) * 0.5, jnp.int32)
    a0 = anchor_ref[0]
    a1 = anchor_ref[1]
    inv_da = lax.rsqrt(a0 * a0 + a1 * a1)
    pk_ref[_OFF_INV:_OFF_INV + 16] = lax.bitcast_convert_type(
        jnp.broadcast_to(inv_da, (16,)), jnp.int32)
    out_copies = []
    for a in range(2):
        for b in range(_B):
            copies[a * _B + b].wait()
            for ch in range(2):
                stag_v[a, b, ch, :, 0] = planes_v[
                    a, b, ch, :, 0:128].reshape(_TY, 8, 128)
                stag_v[a, b, ch, :, 1, :, 0:_H - 128] = planes_v[
                    a, b, ch, :, 128:_H].reshape(_TY, 8, _H - 128)
            oc = pltpu.make_async_copy(stag_v.at[a, b], tab_ref.at[a, b],
                                       sem_out)
            oc.start()
            out_copies.append(oc)
    for oc in out_copies:
        oc.wait()


_repack = pl.pallas_call(
    _repack_kernel,
    out_shape=(
        jax.ShapeDtypeStruct((2, _B, 2, _TY, _TX, 8, 128), jnp.float32),
        jax.ShapeDtypeStruct((_NPACK,), jnp.int32),
    ),
    grid=(1,),
    in_specs=[
        pl.BlockSpec(memory_space=pl.ANY),
        pl.BlockSpec(memory_space=pl.ANY),
        pl.BlockSpec((_B, 2, _NBOX), lambda i: (0, 0, 0)),
        pl.BlockSpec((3, _B, _NNEG), lambda i: (0, 0, 0)),
        pl.BlockSpec((_B, 4, _NBOX), lambda i: (0, 0, 0)),
        pl.BlockSpec((2,), lambda i: (0,)),
    ],
    out_specs=(
        pl.BlockSpec(memory_space=pl.ANY),
        pl.BlockSpec((_NPACK,), lambda i: (0,)),
    ),
    scratch_shapes=[
        pltpu.VMEM((2, _B, 2, _W, _H), jnp.float32),
        pltpu.VMEM((2, _B, 2, _TY, _TX, 8, 128), jnp.float32),
        pltpu.SemaphoreType.DMA((2 * _B,)),
        pltpu.SemaphoreType.DMA,
    ],
)


def _ln(p):
    bits = lax.bitcast_convert_type(p, jnp.int32)
    e = jnp.right_shift(bits, 23) - 127
    m = lax.bitcast_convert_type(
        jnp.bitwise_or(jnp.bitwise_and(bits, 0x007FFFFF), 0x3F800000),
        jnp.float32)
    t = (m - 1.0) / (m + 1.0)
    t2 = t * t
    ln_m = t * (2.0 + t2 * (2.0 / 3.0 + t2 * (2.0 / 5.0
                + t2 * (2.0 / 7.0 + t2 * (2.0 / 9.0)))))
    return e.astype(jnp.float32) * _LN2 + ln_m


def _focal(p):
    one_m = 1.0 - p
    return -_ln(p) * (_ALPHA * one_m * one_m)


def _huber(x):
    ax = jnp.abs(x)
    return jnp.where(ax < 1.0, 0.5 * x * x, ax - 0.5)


def _tile_word(b, y, x):
    plane = b * 2
    tile = (plane * _TY + jnp.right_shift(x, 3)) * _TX + jnp.right_shift(y, 7)
    return (tile * _TILE + jnp.left_shift(jnp.bitwise_and(x, 7), 7)
            + jnp.bitwise_and(y, 127))


_mesh = plsc.VectorSubcoreMesh(core_axis_name="c", subcore_axis_name="s")

_cp = pltpu.CompilerParams()
if "needs_layout_passes" in pltpu.CompilerParams.__dataclass_fields__:
    _cp = dataclasses.replace(_cp, needs_layout_passes=False)


@functools.partial(
    pl.kernel,
    out_type=jax.ShapeDtypeStruct((16,), jnp.float32),
    mesh=_mesh,
    compiler_params=_cp,
    scratch_types=[
        pltpu.VMEM((_NPACK,), jnp.int32),
        pltpu.VMEM((_NIDX,), jnp.int32),
        pltpu.VMEM((_NIDX,), jnp.float32),
        pltpu.VMEM((16,), jnp.float32),
        pltpu.SemaphoreType.DMA,
    ],
)
def _loss_kernel(tab_hbm, pk_hbm, out_hbm,
                 pk_v, idx_v, val_v, out_v, sem):
    cid = lax.axis_index("c")
    sid = lax.axis_index("s")

    @pl.when(jnp.logical_and(cid == 0, sid == 0))
    def _():
        pltpu.sync_copy(pk_hbm, pk_v)

        lanes = lax.iota(jnp.int32, 16)

        for i in range(_NPOS_PAD // 16):
            p = lanes + (i * 16)
            x = pk_v[pl.ds(_OFF_XS + i * 16, 16)]
            y = pk_v[pl.ds(_OFF_YS + i * 16, 16)]
            b = (jnp.where(p >= _NBOX, 1, 0)
                 + jnp.where(p >= 2 * _NBOX, 1, 0)
                 + jnp.where(p >= 3 * _NBOX, 1, 0))
            base = _tile_word(b, y, x)
            idx_v[pl.ds(i * 16, 16)] = base
            idx_v[pl.ds(_NPOS_PAD + i * 16, 16)] = base + _PLANE_W
            idx_v[pl.ds(2 * _NPOS_PAD + i * 16, 16)] = base + 9 * _PLANE_W

        def _stream(off, n):
            return pltpu.async_copy(tab_hbm.at[idx_v.at[pl.ds(off, n)]],
                                    val_v.at[pl.ds(off, n)], sem)

        pos_copies = [_stream(0, 128), _stream(128, 80),
                      _stream(208, 128), _stream(336, 80)]
        car_copies = [_stream(416, 128), _stream(544, 80)]

        for i in range(_NBG_PAD // 16):
            q = lanes + (i * 16)
            bx = pk_v[pl.ds(_OFF_BX + i * 16, 16)]
            by = pk_v[pl.ds(_OFF_BY + i * 16, 16)]
            b = (jnp.where(q >= _NNEG, 1, 0)
                 + jnp.where(q >= 2 * _NNEG, 1, 0)
                 + jnp.where(q >= 3 * _NNEG, 1, 0))
            idx_v[pl.ds(3 * _NPOS_PAD + i * 16, 16)] = (
                _tile_word(b, by, bx) + 8 * _PLANE_W)

        bg_copies = [_stream(624, 128), _stream(752, 128),
                     _stream(880, 128), _stream(1008, 16)]

        inv_da = plsc.bitcast(pk_v[pl.ds(_OFF_INV, 16)], jnp.float32)

        for c in pos_copies:
            c.wait()
        sl_acc = jnp.zeros((16,), jnp.float32)
        for i in range(_NPOS_PAD // 16):
            p = lanes + (i * 16)
            w = jnp.where(p < _NPOS, 1.0, 0.0)
            x_gt = plsc.bitcast(pk_v[pl.ds(_OFF_XG + i * 16, 16)],
                                jnp.float32)
            y_gt = plsc.bitcast(pk_v[pl.ds(_OFF_YG + i * 16, 16)],
                                jnp.float32)
            dx = (x_gt - val_v[pl.ds(i * 16, 16)]) * inv_da
            dy = (y_gt - val_v[pl.ds(_NPOS_PAD + i * 16, 16)]) * inv_da
            sl_acc = sl_acc + w * (_huber(dx) + _huber(dy))

        for c in car_copies:
            c.wait()
        car_acc = jnp.zeros((16,), jnp.float32)
        for i in range(_NPOS_PAD // 16):
            p = lanes + (i * 16)
            w = jnp.where(p < _NPOS, 1.0, 0.0)
            car_acc = car_acc + w * _focal(
                val_v[pl.ds(2 * _NPOS_PAD + i * 16, 16)])

        for c in bg_copies:
            c.wait()
        bg_acc = jnp.zeros((16,), jnp.float32)
        for i in range(_NBG_PAD // 16):
            bg_acc = bg_acc + _focal(val_v[pl.ds(3 * _NPOS_PAD + i * 16, 16)])

        tot = (sl_acc * (_BETA_LOC / _NPOS)
               + car_acc * (1.0 / ((_B - 1) * (_NBOX - 1)))
               + bg_acc * (1.0 / ((_B - 1) * (_NNEG - 1))))
        out_v[...] = jnp.zeros((16,), jnp.float32) + jnp.sum(tot)
        pltpu.sync_copy(out_v, out_hbm)


def kernel(regression_targets, classification_targets_dict, gt_boxes_tensor,
           loc, size, clf, occupancy, angle, heading, anchor):
    loc_t = jnp.swapaxes(loc, 3, 4)
    clf_t = jnp.swapaxes(clf, 3, 4)
    rt_t = jnp.swapaxes(regression_targets.astype(jnp.int32), 1, 2)
    ct_t = jnp.transpose(classification_targets_dict.astype(jnp.int32),
                         (2, 0, 1))
    gt_t = jnp.swapaxes(gt_boxes_tensor.astype(jnp.float32), 1, 2)
    table, packed = _repack(loc_t, clf_t, rt_t, ct_t, gt_t,
                            anchor.astype(jnp.float32))
    out = _loss_kernel(table.reshape(-1), packed)
    return out[0]


# speedup vs baseline: 1.8077x; 1.0653x over previous
import dataclasses
import functools

import jax
import jax.numpy as jnp
from jax import lax
from jax.experimental import pallas as pl
from jax.experimental.pallas import tpu as pltpu
from jax.experimental.pallas import tpu_sc as plsc

_B, _NBOX, _NNEG = 4, 50, 100
_H, _W = 248, 216
_TY, _TX = _W // 8, 2
_TILE = 1024
_PLANE_W = _TY * _TX * _TILE
_NPOS = _B * _NBOX
_NBG = _B * _NNEG
_NPOS_PAD = 208
_NBG_PAD = 400
_NIDX = 3 * _NPOS_PAD + _NBG
_OFF_XS = 0
_OFF_YS = 208
_OFF_BX = 416
_OFF_BY = 816
_OFF_XG = 1216
_OFF_YG = 1424
_OFF_INV = 1632
_NPACK = 1648
_ALPHA = 0.25
_BETA_LOC = 2.0
_LN2 = 0.6931471805599453


def _repack_kernel(loc_ref, clf_ref, rt_ref, ct_ref, gt_ref, anchor_ref,
                   tab_ref, pk_ref, planes_v, stag_v, sems, sem_out):
    copies = []
    for a, ref in enumerate((loc_ref, clf_ref)):
        for b in range(_B):
            copies.append(pltpu.make_async_copy(
                ref.at[b, 0, 0:2], planes_v.at[a, b],
                sems.at[a * _B + b]))
    for c in copies:
        c.start()
    zpad = jnp.zeros((8,), jnp.int32)
    pk_ref[_OFF_XS + _NPOS:_OFF_XS + _NPOS_PAD] = zpad
    pk_ref[_OFF_YS + _NPOS:_OFF_YS + _NPOS_PAD] = zpad
    pk_ref[_OFF_XG + _NPOS:_OFF_XG + _NPOS_PAD] = zpad
    pk_ref[_OFF_YG + _NPOS:_OFF_YG + _NPOS_PAD] = zpad
    for b in range(_B):
        pk_ref[pl.ds(_OFF_XS + b * _NBOX, _NBOX)] = rt_ref[b, 0]
        pk_ref[pl.ds(_OFF_YS + b * _NBOX, _NBOX)] = rt_ref[b, 1]
        pk_ref[pl.ds(_OFF_BX + b * _NNEG, _NNEG)] = ct_ref[1, b]
        pk_ref[pl.ds(_OFF_BY + b * _NNEG, _NNEG)] = ct_ref[2, b]
        g0 = gt_ref[b, 0]
        g1 = gt_ref[b, 1]
        g2 = gt_ref[b, 2]
        g3 = gt_ref[b, 3]
        pk_ref[pl.ds(_OFF_XG + b * _NBOX, _NBOX)] = lax.bitcast_convert_type(
            g0 + (g2 - g0) * 0.5, jnp.int32)
        pk_ref[pl.ds(_OFF_YG + b * _NBOX, _NBOX)] = lax.bitcast_convert_type(
            g1 - (g3 - g1) * 0.5, jnp.int32)
    a0 = anchor_ref[0]
    a1 = anchor_ref[1]
    inv_da = lax.rsqrt(a0 * a0 + a1 * a1)
    pk_ref[_OFF_INV:_OFF_INV + 16] = lax.bitcast_convert_type(
        jnp.broadcast_to(inv_da, (16,)), jnp.int32)
    out_copies = []
    for a in range(2):
        for b in range(_B):
            copies[a * _B + b].wait()
            for ch in range(2):
                stag_v[a, b, ch, :, 0] = planes_v[
                    a, b, ch, :, 0:128].reshape(_TY, 8, 128)
                stag_v[a, b, ch, :, 1, :, 0:_H - 128] = planes_v[
                    a, b, ch, :, 128:_H].reshape(_TY, 8, _H - 128)
            oc = pltpu.make_async_copy(stag_v.at[a, b], tab_ref.at[a, b],
                                       sem_out)
            oc.start()
            out_copies.append(oc)
    for oc in out_copies:
        oc.wait()


_repack = pl.pallas_call(
    _repack_kernel,
    out_shape=(
        jax.ShapeDtypeStruct((2, _B, 2, _TY, _TX, 8, 128), jnp.float32),
        jax.ShapeDtypeStruct((_NPACK,), jnp.int32),
    ),
    grid=(1,),
    in_specs=[
        pl.BlockSpec(memory_space=pl.ANY),
        pl.BlockSpec(memory_space=pl.ANY),
        pl.BlockSpec((_B, 2, _NBOX), lambda i: (0, 0, 0)),
        pl.BlockSpec((3, _B, _NNEG), lambda i: (0, 0, 0)),
        pl.BlockSpec((_B, 4, _NBOX), lambda i: (0, 0, 0)),
        pl.BlockSpec((2,), lambda i: (0,)),
    ],
    out_specs=(
        pl.BlockSpec(memory_space=pl.ANY),
        pl.BlockSpec((_NPACK,), lambda i: (0,)),
    ),
    scratch_shapes=[
        pltpu.VMEM((2, _B, 2, _W, _H), jnp.float32),
        pltpu.VMEM((2, _B, 2, _TY, _TX, 8, 128), jnp.float32),
        pltpu.SemaphoreType.DMA((2 * _B,)),
        pltpu.SemaphoreType.DMA,
    ],
)


def _ln(p):
    bits = lax.bitcast_convert_type(p, jnp.int32)
    e = jnp.right_shift(bits, 23) - 127
    m = lax.bitcast_convert_type(
        jnp.bitwise_or(jnp.bitwise_and(bits, 0x007FFFFF), 0x3F800000),
        jnp.float32)
    t = (m - 1.0) / (m + 1.0)
    t2 = t * t
    ln_m = t * (2.0 + t2 * (2.0 / 3.0 + t2 * (2.0 / 5.0
                + t2 * (2.0 / 7.0 + t2 * (2.0 / 9.0)))))
    return e.astype(jnp.float32) * _LN2 + ln_m


def _focal(p):
    one_m = 1.0 - p
    return -_ln(p) * (_ALPHA * one_m * one_m)


def _huber(x):
    ax = jnp.abs(x)
    return jnp.where(ax < 1.0, 0.5 * x * x, ax - 0.5)


def _tile_word(b, y, x):
    plane = b * 2
    tile = (plane * _TY + jnp.right_shift(x, 3)) * _TX + jnp.right_shift(y, 7)
    return (tile * _TILE + jnp.left_shift(jnp.bitwise_and(x, 7), 7)
            + jnp.bitwise_and(y, 127))


_mesh = plsc.VectorSubcoreMesh(core_axis_name="c", subcore_axis_name="s",
                               num_cores=1)

_cp = pltpu.CompilerParams()
if "needs_layout_passes" in pltpu.CompilerParams.__dataclass_fields__:
    _cp = dataclasses.replace(_cp, needs_layout_passes=False)


@functools.partial(
    pl.kernel,
    out_type=jax.ShapeDtypeStruct((16,), jnp.float32),
    mesh=_mesh,
    compiler_params=_cp,
    scratch_types=[
        pltpu.VMEM((_NPACK,), jnp.int32),
        pltpu.VMEM((_NIDX,), jnp.int32),
        pltpu.VMEM((_NIDX,), jnp.float32),
        pltpu.VMEM((16,), jnp.float32),
        pltpu.SemaphoreType.DMA,
    ],
)
def _loss_kernel(tab_hbm, pk_hbm, out_hbm,
                 pk_v, idx_v, val_v, out_v, sem):
    cid = lax.axis_index("c")
    sid = lax.axis_index("s")

    @pl.when(jnp.logical_and(cid == 0, sid == 0))
    def _():
        pltpu.sync_copy(pk_hbm, pk_v)

        lanes = lax.iota(jnp.int32, 16)

        for i in range(_NPOS_PAD // 16):
            p = lanes + (i * 16)
            x = pk_v[pl.ds(_OFF_XS + i * 16, 16)]
            y = pk_v[pl.ds(_OFF_YS + i * 16, 16)]
            b = (jnp.where(p >= _NBOX, 1, 0)
                 + jnp.where(p >= 2 * _NBOX, 1, 0)
                 + jnp.where(p >= 3 * _NBOX, 1, 0))
            base = _tile_word(b, y, x)
            idx_v[pl.ds(i * 16, 16)] = base
            idx_v[pl.ds(_NPOS_PAD + i * 16, 16)] = base + _PLANE_W
            idx_v[pl.ds(2 * _NPOS_PAD + i * 16, 16)] = base + 9 * _PLANE_W

        def _stream(off, n):
            return pltpu.async_copy(tab_hbm.at[idx_v.at[pl.ds(off, n)]],
                                    val_v.at[pl.ds(off, n)], sem)

        pos_copies = [_stream(0, 128), _stream(128, 80),
                      _stream(208, 128), _stream(336, 80)]
        car_copies = [_stream(416, 128), _stream(544, 80)]

        for i in range(_NBG_PAD // 16):
            q = lanes + (i * 16)
            bx = pk_v[pl.ds(_OFF_BX + i * 16, 16)]
            by = pk_v[pl.ds(_OFF_BY + i * 16, 16)]
            b = (jnp.where(q >= _NNEG, 1, 0)
                 + jnp.where(q >= 2 * _NNEG, 1, 0)
                 + jnp.where(q >= 3 * _NNEG, 1, 0))
            idx_v[pl.ds(3 * _NPOS_PAD + i * 16, 16)] = (
                _tile_word(b, by, bx) + 8 * _PLANE_W)

        bg_copies = [_stream(624, 128), _stream(752, 128),
                     _stream(880, 128), _stream(1008, 16)]

        inv_da = plsc.bitcast(pk_v[pl.ds(_OFF_INV, 16)], jnp.float32)

        for c in pos_copies:
            c.wait()
        sl_acc = jnp.zeros((16,), jnp.float32)
        for i in range(_NPOS_PAD // 16):
            p = lanes + (i * 16)
            w = jnp.where(p < _NPOS, 1.0, 0.0)
            x_gt = plsc.bitcast(pk_v[pl.ds(_OFF_XG + i * 16, 16)],
                                jnp.float32)
            y_gt = plsc.bitcast(pk_v[pl.ds(_OFF_YG + i * 16, 16)],
                                jnp.float32)
            dx = (x_gt - val_v[pl.ds(i * 16, 16)]) * inv_da
            dy = (y_gt - val_v[pl.ds(_NPOS_PAD + i * 16, 16)]) * inv_da
            sl_acc = sl_acc + w * (_huber(dx) + _huber(dy))

        for c in car_copies:
            c.wait()
        car_acc = jnp.zeros((16,), jnp.float32)
        for i in range(_NPOS_PAD // 16):
            p = lanes + (i * 16)
            w = jnp.where(p < _NPOS, 1.0, 0.0)
            car_acc = car_acc + w * _focal(
                val_v[pl.ds(2 * _NPOS_PAD + i * 16, 16)])

        for c in bg_copies:
            c.wait()
        bg_acc = jnp.zeros((16,), jnp.float32)
        for i in range(_NBG_PAD // 16):
            bg_acc = bg_acc + _focal(val_v[pl.ds(3 * _NPOS_PAD + i * 16, 16)])

        tot = (sl_acc * (_BETA_LOC / _NPOS)
               + car_acc * (1.0 / ((_B - 1) * (_NBOX - 1)))
               + bg_acc * (1.0 / ((_B - 1) * (_NNEG - 1))))
        out_v[...] = jnp.zeros((16,), jnp.float32) + jnp.sum(tot)
        pltpu.sync_copy(out_v, out_hbm)


def kernel(regression_targets, classification_targets_dict, gt_boxes_tensor,
           loc, size, clf, occupancy, angle, heading, anchor):
    loc_t = jnp.swapaxes(loc, 3, 4)
    clf_t = jnp.swapaxes(clf, 3, 4)
    rt_t = jnp.swapaxes(regression_targets.astype(jnp.int32), 1, 2)
    ct_t = jnp.transpose(classification_targets_dict.astype(jnp.int32),
                         (2, 0, 1))
    gt_t = jnp.swapaxes(gt_boxes_tensor.astype(jnp.float32), 1, 2)
    table, packed = _repack(loc_t, clf_t, rt_t, ct_t, gt_t,
                            anchor.astype(jnp.float32))
    out = _loss_kernel(table.reshape(-1), packed)
    return out[0]
